# Initial kernel scaffold; baseline (speedup 1.0000x reference)
import jax
import jax.numpy as jnp
from jax.experimental import pallas as pl


def kernel(sp_num_nodes, u_x_dict, u_pos, v_x_dict, v_pos, y_x_dict, y_pos, evt_y, u_plane_u, u_nexus_sp, v_plane_v, v_nexus_sp, y_plane_y, y_nexus_sp, W_enc, b_enc, W_msg, W_self, W_nex, W_back, W_sem, W_filt, W_evt):
    raise NotImplementedError("write your pallas kernel here")


# speedup vs baseline: 2.0174x; 2.0174x over previous
import functools

import jax
import jax.numpy as jnp
import numpy as np
from jax import lax
from jax.experimental import pallas as pl
from jax.experimental.pallas import tpu as pltpu
from jax.experimental.pallas import tpu_sc as plsc

N_NODE = 50000
SP_NN = 50000
E_PL = 800000
E_NX = 100000
HID = 64
NSUB = 16
K = 80
H_HALF = N_NODE // 2
TRASH = H_HALF
ACC_ROWS = H_HALF + 120
OUT_SL = 1000

_NORM = {
    'u': np.array([[389.42752, 172.90794, 147.81108, 4.5563765], [147.1627, 78.01324, 228.31424, 2.2156637]], dtype=np.float32),
    'v': np.array([[368.83023, 173.01247, 154.14513, 4.449338], [145.29645, 80.54078, 282.34027, 1.8969047]], dtype=np.float32),
    'y': np.array([[546.2973, 172.77615, 116.974, 4.1647816], [283.47656, 73.99135, 115.49256, 1.4615369]], dtype=np.float32),
}


def _zero_zbuf(zbuf):
    def row(i, carry):
        for j in range(HID // 16):
            zbuf[i, pl.ds(j * 16, 16)] = jnp.zeros((16,), jnp.float32)
        return carry
    lax.fori_loop(0, K, row, 0)


def _zero_acc(acc, zbuf, s):
    nslices = ACC_ROWS // K
    cnt = (nslices - s + NSUB - 1) // NSUB

    def body(k, carry):
        sl = s + k * NSUB
        pltpu.sync_copy(zbuf, acc.at[pl.ds(sl * K, K)])
        return carry
    lax.fori_loop(0, cnt, body, 0)


def _accum(table, src_hbm, dst_hbm, src_v, dst_v, rows_v, acc, sem, s, lo, hi, E):
    G = E // K
    cnt = (G - s + NSUB - 1) // NSUB
    iota = lax.iota(jnp.int32, 16)

    def body(k, carry):
        base = pl.multiple_of((s + k * NSUB) * K, 8)
        pltpu.sync_copy(src_hbm.at[pl.ds(base, K)], src_v)
        pltpu.sync_copy(dst_hbm.at[pl.ds(base, K)], dst_v)
        for i in range(K // 16):
            d = dst_v[pl.ds(i * 16, 16)]
            inr = (d >= lo) & (d < hi)
            d2 = jnp.where(inr, d - lo, TRASH + i * 16 + iota)
            dst_v[pl.ds(i * 16, 16)] = d2
        pltpu.async_copy(table.at[src_v], rows_v, sem).wait()
        pltpu.sync_copy(rows_v, acc.at[dst_v], add=True)
        return carry
    lax.fori_loop(0, cnt, body, 0)


def _writeout(acc, out_hbm, s, lo):
    for t in range(2):
        sl = s + NSUB * t

        @pl.when(sl < H_HALF // OUT_SL)
        def _():
            pltpu.sync_copy(acc.at[pl.ds(sl * OUT_SL, OUT_SL)],
                            out_hbm.at[pl.ds(lo + sl * OUT_SL, OUT_SL)])


def _seg1_body(E):
    def body(table, src_hbm, dst_hbm, out_hbm, src_v, dst_v, rows_v, zbuf, acc, sem):
        c = lax.axis_index("c")
        s = lax.axis_index("s")
        lo = c * H_HALF
        hi = lo + H_HALF
        _zero_zbuf(zbuf)
        _zero_acc(acc, zbuf, s)
        plsc.subcore_barrier()
        _accum(table, src_hbm, dst_hbm, src_v, dst_v, rows_v, acc, sem, s, lo, hi, E)
        plsc.subcore_barrier()
        _writeout(acc, out_hbm, s, lo)
    return body


def _seg3_body(E):
    def body(t0, t1, t2, s0, s1, s2, d0, d1, d2, out_hbm,
             src_v, dst_v, rows_v, zbuf, acc, sem):
        c = lax.axis_index("c")
        s = lax.axis_index("s")
        lo = c * H_HALF
        hi = lo + H_HALF
        _zero_zbuf(zbuf)
        _zero_acc(acc, zbuf, s)
        plsc.subcore_barrier()
        for table, src_hbm, dst_hbm in ((t0, s0, d0), (t1, s1, d1), (t2, s2, d2)):
            _accum(table, src_hbm, dst_hbm, src_v, dst_v, rows_v, acc, sem, s, lo, hi, E)
        plsc.subcore_barrier()
        _writeout(acc, out_hbm, s, lo)
    return body


def _sc_scratch():
    return [
        pltpu.VMEM((K,), jnp.int32),
        pltpu.VMEM((K,), jnp.int32),
        pltpu.VMEM((K, HID), jnp.float32),
        pltpu.VMEM((K, HID), jnp.float32),
        pltpu.VMEM_SHARED((ACC_ROWS, HID), jnp.float32),
        pltpu.SemaphoreType.DMA,
    ]


def _mesh():
    return plsc.VectorSubcoreMesh(core_axis_name="c", subcore_axis_name="s",
                                  num_cores=2, num_subcores=NSUB)


_SC_PARAMS = pltpu.CompilerParams(use_tc_tiling_on_sc=False)


@functools.cache
def _seg1(E):
    mesh = _mesh()
    return pl.kernel(
        _seg1_body(E),
        out_type=jax.ShapeDtypeStruct((N_NODE, HID), jnp.float32),
        mesh=mesh,
        scratch_types=_sc_scratch(),
        compiler_params=_SC_PARAMS,
    )


@functools.cache
def _seg3(E):
    mesh = _mesh()
    return pl.kernel(
        _seg3_body(E),
        out_type=jax.ShapeDtypeStruct((N_NODE, HID), jnp.float32),
        mesh=mesh,
        scratch_types=_sc_scratch(),
        compiler_params=_SC_PARAMS,
    )


_BR = 2000
_GRID = N_NODE // _BR


def _rows(d):
    return pl.BlockSpec((_BR, d), lambda i: (i, 0))


def _whole(shape):
    return pl.BlockSpec(shape, lambda i: tuple(0 for _ in shape))


def _enc_body(x_ref, p_ref, mu_ref, sd_ref, w1_ref, w2_ref, b_ref, o_ref):
    xn = (x_ref[...] - mu_ref[...]) / sd_ref[...]
    o = (jnp.dot(xn, w1_ref[...], preferred_element_type=jnp.float32)
         + jnp.dot(p_ref[...], w2_ref[...], preferred_element_type=jnp.float32)
         + b_ref[...])
    o_ref[...] = jnp.tanh(o)


@jax.jit
def _enc(x, pos, mu, sd, w1, w2, b):
    return pl.pallas_call(
        _enc_body,
        grid=(_GRID,),
        in_specs=[_rows(4), _rows(2), _whole((1, 4)), _whole((1, 4)),
                  _whole((4, HID)), _whole((2, HID)), _whole((1, HID))],
        out_specs=_rows(HID),
        out_shape=jax.ShapeDtypeStruct((N_NODE, HID), jnp.float32),
    )(x, pos, mu, sd, w1, w2, b)


def _upd_body(h_ref, a_ref, ws_ref, wm_ref, o_ref):
    o = (jnp.dot(h_ref[...], ws_ref[...], preferred_element_type=jnp.float32)
         + jnp.dot(a_ref[...], wm_ref[...], preferred_element_type=jnp.float32))
    o_ref[...] = jnp.tanh(o)


@jax.jit
def _upd(h, agg, ws, wm):
    return pl.pallas_call(
        _upd_body,
        grid=(_GRID,),
        in_specs=[_rows(HID), _rows(HID), _whole((HID, HID)), _whole((HID, HID))],
        out_specs=_rows(HID),
        out_shape=jax.ShapeDtypeStruct((N_NODE, HID), jnp.float32),
    )(h, agg, ws, wm)


def _spupd_body(spz_ref, p_ref, wn_ref, o_ref):
    o = jnp.dot(p_ref[...], wn_ref[...], preferred_element_type=jnp.float32)
    o_ref[...] = jnp.tanh(o + spz_ref[0])


@jax.jit
def _spupd(spz, pre, wn):
    return pl.pallas_call(
        _spupd_body,
        grid=(_GRID,),
        in_specs=[pl.BlockSpec(memory_space=pltpu.SMEM),
                  _rows(HID), _whole((HID, HID))],
        out_specs=_rows(HID),
        out_shape=jax.ShapeDtypeStruct((SP_NN, HID), jnp.float32),
    )(spz, pre, wn)


def _back_body(h_ref, b_ref, wb_ref, o_ref):
    o = h_ref[...] + jnp.dot(b_ref[...], wb_ref[...], preferred_element_type=jnp.float32)
    o_ref[...] = jnp.tanh(o)


@jax.jit
def _backupd(h, back, wb):
    return pl.pallas_call(
        _back_body,
        grid=(_GRID,),
        in_specs=[_rows(HID), _rows(HID), _whole((HID, HID))],
        out_specs=_rows(HID),
        out_shape=jax.ShapeDtypeStruct((N_NODE, HID), jnp.float32),
    )(h, back, wb)


def _heads_body(h_ref, w6_ref, o_ref):
    o_ref[...] = jnp.dot(h_ref[...], w6_ref[...], preferred_element_type=jnp.float32)


@jax.jit
def _heads(h, w6):
    return pl.pallas_call(
        _heads_body,
        grid=(_GRID,),
        in_specs=[_rows(HID), _whole((HID, 6))],
        out_specs=_rows(6),
        out_shape=jax.ShapeDtypeStruct((N_NODE, 6), jnp.float32),
    )(h, w6)


def _evt_body(sp_ref, we_ref, e_ref, v_ref):
    i = pl.program_id(0)

    @pl.when(i == 0)
    def _():
        e_ref[...] = jnp.zeros_like(e_ref)

    e_ref[...] += jnp.sum(sp_ref[...], axis=0, keepdims=True)

    @pl.when(i == _GRID - 1)
    def _():
        e = e_ref[...] / np.float32(SP_NN)
        e_ref[...] = e
        v_ref[...] = jnp.dot(e, we_ref[...], preferred_element_type=jnp.float32)


@jax.jit
def _evt(sp, we):
    return pl.pallas_call(
        _evt_body,
        grid=(_GRID,),
        in_specs=[_rows(HID), _whole((HID, 5))],
        out_specs=[_whole((1, HID)), _whole((1, 5))],
        out_shape=[jax.ShapeDtypeStruct((1, HID), jnp.float32),
                   jax.ShapeDtypeStruct((1, 5), jnp.float32)],
    )(sp, we)


def kernel(sp_num_nodes, u_x_dict, u_pos, v_x_dict, v_pos, y_x_dict, y_pos, evt_y,
           u_plane_u, u_nexus_sp, v_plane_v, v_nexus_sp, y_plane_y, y_nexus_sp,
           W_enc, b_enc, W_msg, W_self, W_nex, W_back, W_sem, W_filt, W_evt):
    f32, i32 = jnp.float32, jnp.int32
    planes = ('u', 'v', 'y')
    xs = {'u': u_x_dict, 'v': v_x_dict, 'y': y_x_dict}
    poss = {'u': u_pos, 'v': v_pos, 'y': y_pos}
    pe = {'u': u_plane_u.astype(i32), 'v': v_plane_v.astype(i32), 'y': y_plane_y.astype(i32)}
    ne = {'u': u_nexus_sp.astype(i32), 'v': v_nexus_sp.astype(i32), 'y': y_nexus_sp.astype(i32)}

    spz = (sp_num_nodes[0] - SP_NN).astype(f32).reshape(1)
    w2 = W_enc[4:6]
    b2 = b_enc.reshape(1, HID)

    h = {}
    for p in planes:
        mu = jnp.asarray(_NORM[p][0]).reshape(1, 4)
        sd = jnp.asarray(_NORM[p][1]).reshape(1, 4)
        h[p] = _enc(xs[p], poss[p], mu, sd, W_enc[:4], w2, b2)

    ps = {p: pe[p][0] for p in planes}
    pd = {p: pe[p][1] for p in planes}
    ns = {p: ne[p][0] for p in planes}
    nd = {p: ne[p][1] for p in planes}

    for _ in range(3):
        agg = {p: _seg1(E_PL)(h[p], ps[p], pd[p]) for p in planes}
        h = {p: _upd(h[p], agg[p], W_self, W_msg) for p in planes}
        pre = _seg3(E_NX)(h['u'], h['v'], h['y'],
                          ns['u'], ns['v'], ns['y'],
                          nd['u'], nd['v'], nd['y'])
        sp = _spupd(spz, pre, W_nex)
        back = {p: _seg1(E_NX)(sp, nd[p], ns[p]) for p in planes}
        h = {p: _backupd(h[p], back[p], W_back) for p in planes}

    w6 = jnp.concatenate([W_sem, W_filt], axis=1)
    x6 = {p: _heads(h[p], w6) for p in planes}
    e_evt, v_evt = _evt(sp, W_evt)
    return (e_evt,
            x6['u'][:, :5], x6['v'][:, :5], x6['y'][:, :5],
            x6['u'][:, 5], x6['v'][:, 5], x6['y'][:, 5],
            v_evt)


# speedup vs baseline: 5.3041x; 2.6292x over previous
import functools

import jax
import jax.numpy as jnp
import numpy as np
from jax import lax
from jax.experimental import pallas as pl
from jax.experimental.pallas import tpu as pltpu
from jax.experimental.pallas import tpu_sc as plsc

N_NODE = 50000
SP_NN = 50000
E_PL = 800000
E_NX = 100000
HID = 64
FH = HID // 2
NSUB = 16

_CHUNK = {E_PL: (128, 10), E_NX: (80, 10)}

_ZS = 625


def _norm_np():
    return {
        'u': np.array([[389.42752, 172.90794, 147.81108, 4.5563765], [147.1627, 78.01324, 228.31424, 2.2156637]], dtype=np.float32),
        'v': np.array([[368.83023, 173.01247, 154.14513, 4.449338], [145.29645, 80.54078, 282.34027, 1.8969047]], dtype=np.float32),
        'y': np.array([[546.2973, 172.77615, 116.974, 4.1647816], [283.47656, 73.99135, 115.49256, 1.4615369]], dtype=np.float32),
    }


_NORM = _norm_np()


def _zero_zbuf(zbuf):
    def row(i, carry):
        for j in range(FH // 16):
            zbuf[i, pl.ds(j * 16, 16)] = jnp.zeros((16,), jnp.float32)
        return carry
    lax.fori_loop(0, _ZS, row, 0)


def _zero_acc(acc, zbuf, s):
    for t in range(N_NODE // _ZS // NSUB):
        sl = s * (N_NODE // _ZS // NSUB) + t
        pltpu.sync_copy(zbuf, acc.at[pl.ds(sl * _ZS, _ZS)])


def _accum(table, src2, dst2, src_vs, dst_vs, rows, acc, sems, s, E):
    K, S = _CHUNK[E]
    nsuper = E // (K * S)
    cnt = (nsuper - s + NSUB - 1) // NSUB

    def body(k, carry):
        sg = s + k * NSUB
        pltpu.sync_copy(src2.at[pl.ds(sg * S, S)], src_vs)
        pltpu.sync_copy(dst2.at[pl.ds(sg * S, S)], dst_vs)
        descs = [None, None]
        descs[0] = pltpu.async_copy(table.at[src_vs.at[0]], rows[0], sems[0])
        for j in range(S):
            b = j & 1
            if j + 1 < S:
                descs[b ^ 1] = pltpu.async_copy(
                    table.at[src_vs.at[j + 1]], rows[b ^ 1], sems[b ^ 1])
            descs[b].wait()
            pltpu.sync_copy(rows[b], acc.at[dst_vs.at[j]], add=True)
        return carry
    lax.fori_loop(0, cnt, body, 0)


def _writeout(acc, out_hbm, s):
    n = N_NODE // NSUB
    pltpu.sync_copy(acc.at[pl.ds(s * n, n)], out_hbm.at[pl.ds(s * n, n)])


def _seg1_body(E):
    def body(t_lo, t_hi, src2, dst2, out_lo, out_hi,
             src_vs, dst_vs, r0, r1, zbuf, acc, sem0, sem1):
        c = lax.axis_index("c")
        s = lax.axis_index("s")
        _zero_zbuf(zbuf)
        _zero_acc(acc, zbuf, s)
        plsc.subcore_barrier()

        @pl.when(c == 0)
        def _():
            _accum(t_lo, src2, dst2, src_vs, dst_vs, (r0, r1), acc, (sem0, sem1), s, E)

        @pl.when(c == 1)
        def _():
            _accum(t_hi, src2, dst2, src_vs, dst_vs, (r0, r1), acc, (sem0, sem1), s, E)

        plsc.subcore_barrier()

        @pl.when(c == 0)
        def _():
            _writeout(acc, out_lo, s)

        @pl.when(c == 1)
        def _():
            _writeout(acc, out_hi, s)
    return body


def _seg3_body(E):
    def body(tu_lo, tu_hi, tv_lo, tv_hi, ty_lo, ty_hi,
             s0, s1, s2, d0, d1, d2, out_lo, out_hi,
             src_vs, dst_vs, r0, r1, zbuf, acc, sem0, sem1):
        c = lax.axis_index("c")
        s = lax.axis_index("s")
        _zero_zbuf(zbuf)
        _zero_acc(acc, zbuf, s)
        plsc.subcore_barrier()

        @pl.when(c == 0)
        def _():
            for t, sr, dr in ((tu_lo, s0, d0), (tv_lo, s1, d1), (ty_lo, s2, d2)):
                _accum(t, sr, dr, src_vs, dst_vs, (r0, r1), acc, (sem0, sem1), s, E)

        @pl.when(c == 1)
        def _():
            for t, sr, dr in ((tu_hi, s0, d0), (tv_hi, s1, d1), (ty_hi, s2, d2)):
                _accum(t, sr, dr, src_vs, dst_vs, (r0, r1), acc, (sem0, sem1), s, E)

        plsc.subcore_barrier()

        @pl.when(c == 0)
        def _():
            _writeout(acc, out_lo, s)

        @pl.when(c == 1)
        def _():
            _writeout(acc, out_hi, s)
    return body


def _sc_scratch(E):
    K, S = _CHUNK[E]
    return [
        pltpu.VMEM((S, K), jnp.int32),
        pltpu.VMEM((S, K), jnp.int32),
        pltpu.VMEM((K, FH), jnp.float32),
        pltpu.VMEM((K, FH), jnp.float32),
        pltpu.VMEM((_ZS, FH), jnp.float32),
        pltpu.VMEM_SHARED((N_NODE, FH), jnp.float32),
        pltpu.SemaphoreType.DMA,
        pltpu.SemaphoreType.DMA,
    ]


def _mesh():
    return plsc.VectorSubcoreMesh(core_axis_name="c", subcore_axis_name="s",
                                  num_cores=2, num_subcores=NSUB)


_SC_PARAMS = pltpu.CompilerParams(use_tc_tiling_on_sc=False)

_HALF = jax.ShapeDtypeStruct((N_NODE, FH), jnp.float32)


@functools.cache
def _seg1(E):
    return pl.kernel(
        _seg1_body(E),
        out_type=(_HALF, _HALF),
        mesh=_mesh(),
        scratch_types=_sc_scratch(E),
        compiler_params=_SC_PARAMS,
    )


@functools.cache
def _seg3(E):
    return pl.kernel(
        _seg3_body(E),
        out_type=(_HALF, _HALF),
        mesh=_mesh(),
        scratch_types=_sc_scratch(E),
        compiler_params=_SC_PARAMS,
    )


_BR = 2000
_GRID = N_NODE // _BR


def _rows(d):
    return pl.BlockSpec((_BR, d), lambda i: (i, 0))


def _whole(shape):
    return pl.BlockSpec(shape, lambda i: tuple(0 for _ in shape))


def _split_store(o, lo_ref, hi_ref):
    lo_ref[...] = o[:, :FH]
    hi_ref[...] = o[:, FH:]


def _enc_body(x_ref, p_ref, mu_ref, sd_ref, we_ref, b_ref, wm_ref,
              lo_ref, hi_ref, mlo_ref, mhi_ref):
    xn = (x_ref[...] - mu_ref[...]) / sd_ref[...]
    f = jnp.concatenate([xn, p_ref[...]], axis=1)
    h = jnp.tanh(jnp.dot(f, we_ref[...], preferred_element_type=jnp.float32)
                 + b_ref[...])
    _split_store(h, lo_ref, hi_ref)
    _split_store(jnp.dot(h, wm_ref[...], preferred_element_type=jnp.float32),
                 mlo_ref, mhi_ref)


@jax.jit
def _enc(x, pos, mu, sd, we, b, wm):
    return pl.pallas_call(
        _enc_body,
        grid=(_GRID,),
        in_specs=[_rows(4), _rows(2), _whole((1, 4)), _whole((1, 4)),
                  _whole((6, HID)), _whole((1, HID)), _whole((HID, HID))],
        out_specs=[_rows(FH)] * 4,
        out_shape=[_HALF] * 4,
    )(x, pos, mu, sd, we, b, wm)


def _upd_body(hl_ref, hh_ref, al_ref, ah_ref, ws_ref, wn_ref,
              lo_ref, hi_ref, nlo_ref, nhi_ref):
    h = jnp.concatenate([hl_ref[...], hh_ref[...]], axis=1)
    a = jnp.concatenate([al_ref[...], ah_ref[...]], axis=1)
    nh = jnp.tanh(jnp.dot(h, ws_ref[...], preferred_element_type=jnp.float32) + a)
    _split_store(nh, lo_ref, hi_ref)
    _split_store(jnp.dot(nh, wn_ref[...], preferred_element_type=jnp.float32),
                 nlo_ref, nhi_ref)


@jax.jit
def _upd(hl, hh, al, ah, ws, wn):
    return pl.pallas_call(
        _upd_body,
        grid=(_GRID,),
        in_specs=[_rows(FH)] * 4 + [_whole((HID, HID))] * 2,
        out_specs=[_rows(FH)] * 4,
        out_shape=[_HALF] * 4,
    )(hl, hh, al, ah, ws, wn)


def _spupd_body(spz_ref, pl_ref, ph_ref, wb_ref,
                lo_ref, hi_ref, blo_ref, bhi_ref):
    p = jnp.concatenate([pl_ref[...], ph_ref[...]], axis=1)
    sp = jnp.tanh(p + spz_ref[0])
    _split_store(sp, lo_ref, hi_ref)
    _split_store(jnp.dot(sp, wb_ref[...], preferred_element_type=jnp.float32),
                 blo_ref, bhi_ref)


@jax.jit
def _spupd(spz, prel, preh, wb):
    return pl.pallas_call(
        _spupd_body,
        grid=(_GRID,),
        in_specs=[pl.BlockSpec(memory_space=pltpu.SMEM),
                  _rows(FH), _rows(FH), _whole((HID, HID))],
        out_specs=[_rows(FH)] * 4,
        out_shape=[_HALF] * 4,
    )(spz, prel, preh, wb)


def _back_body(hl_ref, hh_ref, bl_ref, bh_ref, wm_ref,
               lo_ref, hi_ref, mlo_ref, mhi_ref):
    h = jnp.concatenate([hl_ref[...], hh_ref[...]], axis=1)
    b = jnp.concatenate([bl_ref[...], bh_ref[...]], axis=1)
    nh = jnp.tanh(h + b)
    _split_store(nh, lo_ref, hi_ref)
    _split_store(jnp.dot(nh, wm_ref[...], preferred_element_type=jnp.float32),
                 mlo_ref, mhi_ref)


@jax.jit
def _backupd(hl, hh, bl, bh, wm):
    return pl.pallas_call(
        _back_body,
        grid=(_GRID,),
        in_specs=[_rows(FH)] * 4 + [_whole((HID, HID))],
        out_specs=[_rows(FH)] * 4,
        out_shape=[_HALF] * 4,
    )(hl, hh, bl, bh, wm)


def _heads_body(hl_ref, hh_ref, w6_ref, o_ref):
    h = jnp.concatenate([hl_ref[...], hh_ref[...]], axis=1)
    o_ref[...] = jnp.dot(h, w6_ref[...], preferred_element_type=jnp.float32)


@jax.jit
def _heads(hl, hh, w6):
    return pl.pallas_call(
        _heads_body,
        grid=(_GRID,),
        in_specs=[_rows(FH), _rows(FH), _whole((HID, 6))],
        out_specs=_rows(6),
        out_shape=jax.ShapeDtypeStruct((N_NODE, 6), jnp.float32),
    )(hl, hh, w6)


def _evt_body(sl_ref, sh_ref, we_ref, e_ref, v_ref):
    i = pl.program_id(0)

    @pl.when(i == 0)
    def _():
        e_ref[...] = jnp.zeros_like(e_ref)

    sp = jnp.concatenate([sl_ref[...], sh_ref[...]], axis=1)
    e_ref[...] += jnp.sum(sp, axis=0, keepdims=True)

    @pl.when(i == _GRID - 1)
    def _():
        e = e_ref[...] / np.float32(SP_NN)
        e_ref[...] = e
        v_ref[...] = jnp.dot(e, we_ref[...], preferred_element_type=jnp.float32)


@jax.jit
def _evt(sl, sh, we):
    return pl.pallas_call(
        _evt_body,
        grid=(_GRID,),
        in_specs=[_rows(FH), _rows(FH), _whole((HID, 5))],
        out_specs=[_whole((1, HID)), _whole((1, 5))],
        out_shape=[jax.ShapeDtypeStruct((1, HID), jnp.float32),
                   jax.ShapeDtypeStruct((1, 5), jnp.float32)],
    )(sl, sh, we)


def kernel(sp_num_nodes, u_x_dict, u_pos, v_x_dict, v_pos, y_x_dict, y_pos, evt_y,
           u_plane_u, u_nexus_sp, v_plane_v, v_nexus_sp, y_plane_y, y_nexus_sp,
           W_enc, b_enc, W_msg, W_self, W_nex, W_back, W_sem, W_filt, W_evt):
    f32, i32 = jnp.float32, jnp.int32
    planes = ('u', 'v', 'y')
    xs = {'u': u_x_dict, 'v': v_x_dict, 'y': y_x_dict}
    poss = {'u': u_pos, 'v': v_pos, 'y': y_pos}
    pe = {'u': u_plane_u.astype(i32), 'v': v_plane_v.astype(i32), 'y': y_plane_y.astype(i32)}
    ne = {'u': u_nexus_sp.astype(i32), 'v': v_nexus_sp.astype(i32), 'y': y_nexus_sp.astype(i32)}

    spz = (sp_num_nodes[0] - SP_NN).astype(f32).reshape(1)
    w2 = W_enc[4:6]
    b2 = b_enc.reshape(1, HID)

    mu = {p: jnp.asarray(_NORM[p][0]).reshape(1, 4) for p in planes}
    sd = {p: jnp.asarray(_NORM[p][1]).reshape(1, 4) for p in planes}
    hm = {}
    for p in planes:
        hm[p] = _enc(xs[p], poss[p], mu[p], sd[p], W_enc, b2, W_msg)

    Kp, Sp = _CHUNK[E_PL]
    Kn, Sn = _CHUNK[E_NX]
    ps = {p: pe[p][0].reshape(E_PL // Kp, Kp) for p in planes}
    pd = {p: pe[p][1].reshape(E_PL // Kp, Kp) for p in planes}
    ns = {p: ne[p][0].reshape(E_NX // Kn, Kn) for p in planes}
    nd = {p: ne[p][1].reshape(E_NX // Kn, Kn) for p in planes}

    for _ in range(3):
        agg = {p: _seg1(E_PL)(hm[p][2], hm[p][3], ps[p], pd[p]) for p in planes}
        hn = {p: _upd(hm[p][0], hm[p][1], agg[p][0], agg[p][1], W_self, W_nex)
              for p in planes}
        pre = _seg3(E_NX)(hn['u'][2], hn['u'][3], hn['v'][2], hn['v'][3],
                          hn['y'][2], hn['y'][3],
                          ns['u'], ns['v'], ns['y'],
                          nd['u'], nd['v'], nd['y'])
        sp = _spupd(spz, pre[0], pre[1], W_back)
        back = {p: _seg1(E_NX)(sp[2], sp[3], nd[p], ns[p]) for p in planes}
        hm = {p: _backupd(hn[p][0], hn[p][1], back[p][0], back[p][1], W_msg)
              for p in planes}

    w6 = jnp.concatenate([W_sem, W_filt], axis=1)
    x6 = {p: _heads(hm[p][0], hm[p][1], w6) for p in planes}
    e_evt, v_evt = _evt(sp[0], sp[1], W_evt)
    return (e_evt,
            x6['u'][:, :5], x6['v'][:, :5], x6['y'][:, :5],
            x6['u'][:, 5], x6['v'][:, 5], x6['y'][:, 5],
            v_evt)
